# baseline (speedup 1.0000x reference)
import functools

import jax
import jax.numpy as jnp
from jax import lax
from jax.experimental import pallas as pl
from jax.experimental.pallas import tpu as pltpu

QUES = 2048
SEQH = 2048
T = 8192
K = 32

G1 = 8
QROWS = T // G1
WROWS = (3 * SEQH) // G1


def _matvec_body(sel_ref, q_ref, h_ref, ques_ref, wih_ref, whh_ref,
                 alpha_ref, gi_ref, gh_ref):
    q = q_ref[...]
    h = h_ref[...]
    alpha_ref[...] = jnp.dot(ques_ref[...], q,
                             preferred_element_type=jnp.float32)
    gi_ref[...] = jnp.dot(wih_ref[...], q,
                          preferred_element_type=jnp.float32)
    gh_ref[...] = jnp.dot(whh_ref[...], h,
                          preferred_element_type=jnp.float32)


def _topk_body(alpha_ref, idx_ref, w_ref):
    a = alpha_ref[...]
    iota = (lax.broadcasted_iota(jnp.int32, (64, 128), 0) * 128
            + lax.broadcasted_iota(jnp.int32, (64, 128), 1))
    kiota = lax.broadcasted_iota(jnp.int32, (1, K), 1)
    neg = jnp.float32(-jnp.inf)

    def step(j, carry):
        a, idxs, vals = carry
        m = jnp.max(a)
        idx = jnp.min(jnp.where(a == m, iota, T))
        idxs = jnp.where(kiota == j, idx, idxs)
        vals = jnp.where(kiota == j, m, vals)
        a = jnp.where(iota == idx, neg, a)
        return a, idxs, vals

    idxs0 = jnp.zeros((1, K), jnp.int32)
    vals0 = jnp.full((1, K), neg, jnp.float32)
    _, idxs, vals = lax.fori_loop(0, K, step, (a, idxs0, vals0))
    e = jnp.exp(vals - jnp.max(vals))
    w = e / jnp.sum(e)
    idx_ref[...] = idxs
    w_ref[...] = w


def _final_body(idx_ref, w_ref, row_ref, q_ref, ws_ref, bs_ref,
                gi_ref, gh_ref, h_ref, bih_ref, bhh_ref,
                pred_ref, hnew_ref, acc_ref):
    i = pl.program_id(0)

    @pl.when(i == 0)
    def _():
        acc_ref[...] = jnp.zeros_like(acc_ref)

    kiota = lax.broadcasted_iota(jnp.int32, (1, K), 1)
    wi = jnp.sum(jnp.where(kiota == i, w_ref[...], 0.0))
    acc_ref[...] += wi * row_ref[0]

    @pl.when(i == K - 1)
    def _():
        ws = ws_ref[...]
        pred = (jnp.sum(ws[0:1] * q_ref[...])
                + jnp.sum(ws[1:2] * acc_ref[...]) + bs_ref[0, 0])
        pred_ref[...] = pred[None, None]
        gi = gi_ref[...] + bih_ref[...]
        gh = gh_ref[...] + bhh_ref[...]
        h = h_ref[...]
        r = jax.nn.sigmoid(gi[0:16] + gh[0:16])
        z = jax.nn.sigmoid(gi[16:32] + gh[16:32])
        n = jnp.tanh(gi[32:48] + r * gh[32:48])
        hnew_ref[...] = (1.0 - z) * n + z * h


def kernel(question, score, questions, hs, Ws, bs, W_ih, W_hh, b_ih, b_hh):
    f32 = jnp.float32
    q2 = question.reshape(QUES, 1)
    h_prev = hs[T - 1, 0]
    h2 = h_prev.reshape(SEQH, 1)
    sel = (score[0] < 0.5).astype(jnp.int32).reshape(1)

    grid_spec = pltpu.PrefetchScalarGridSpec(
        num_scalar_prefetch=1,
        grid=(G1,),
        in_specs=[
            pl.BlockSpec((QUES, 1), lambda i, s: (0, 0)),
            pl.BlockSpec((SEQH, 1), lambda i, s: (0, 0)),
            pl.BlockSpec((QROWS, QUES), lambda i, s: (i, 0)),
            pl.BlockSpec((WROWS, QUES), lambda i, s: (i, s[0])),
            pl.BlockSpec((WROWS, SEQH), lambda i, s: (i, 0)),
        ],
        out_specs=[
            pl.BlockSpec((QROWS, 1), lambda i, s: (i, 0)),
            pl.BlockSpec((WROWS, 1), lambda i, s: (i, 0)),
            pl.BlockSpec((WROWS, 1), lambda i, s: (i, 0)),
        ],
    )
    alpha, gi, gh = pl.pallas_call(
        _matvec_body,
        grid_spec=grid_spec,
        out_shape=[
            jax.ShapeDtypeStruct((T, 1), f32),
            jax.ShapeDtypeStruct((3 * SEQH, 1), f32),
            jax.ShapeDtypeStruct((3 * SEQH, 1), f32),
        ],
    )(sel, q2, h2, questions, W_ih, W_hh)

    idx, w = pl.pallas_call(
        _topk_body,
        out_shape=[
            jax.ShapeDtypeStruct((1, K), jnp.int32),
            jax.ShapeDtypeStruct((1, K), f32),
        ],
    )(alpha.reshape(T // 128, 128))

    pred, h_new = pl.pallas_call(
        _final_body,
        grid_spec=pltpu.PrefetchScalarGridSpec(
            num_scalar_prefetch=1,
            grid=(K,),
            in_specs=[
                pl.BlockSpec((1, K), lambda i, s: (0, 0)),
                pl.BlockSpec((1, 1, SEQH), lambda i, s: (s[i], 0, 0)),
                pl.BlockSpec((1, QUES), lambda i, s: (0, 0)),
                pl.BlockSpec((2, QUES), lambda i, s: (0, 0)),
                pl.BlockSpec((1, 1), lambda i, s: (0, 0)),
                pl.BlockSpec((48, 128), lambda i, s: (0, 0)),
                pl.BlockSpec((48, 128), lambda i, s: (0, 0)),
                pl.BlockSpec((16, 128), lambda i, s: (0, 0)),
                pl.BlockSpec((48, 128), lambda i, s: (0, 0)),
                pl.BlockSpec((48, 128), lambda i, s: (0, 0)),
            ],
            out_specs=[
                pl.BlockSpec((1, 1), lambda i, s: (0, 0)),
                pl.BlockSpec((16, 128), lambda i, s: (0, 0)),
            ],
            scratch_shapes=[
                pltpu.VMEM((1, SEQH), f32),
            ],
        ),
        out_shape=[
            jax.ShapeDtypeStruct((1, 1), f32),
            jax.ShapeDtypeStruct((16, 128), f32),
        ],
    )(
        idx.reshape(K), w, hs,
        question.reshape(1, QUES), Ws.reshape(2, QUES), bs.reshape(1, 1),
        gi.reshape(48, 128), gh.reshape(48, 128), h_prev.reshape(16, 128),
        b_ih.reshape(48, 128), b_hh.reshape(48, 128),
    )
    return (pred.reshape(1), h_new.reshape(1, 1, SEQH))
